# Initial kernel scaffold; baseline (speedup 1.0000x reference)
import functools

import jax
import jax.numpy as jnp
from jax import lax
from jax.experimental import pallas as pl
from jax.experimental.pallas import tpu as pltpu
from jax.experimental.pallas import tpu_sc as plsc

NC = 2
NS = 16
L = 16
NW = NC * NS
B = 128
NUM_LAYERS = 3


def _round_up(v, m):
    return (v + m - 1) // m * m


def _mesh():
    return plsc.VectorSubcoreMesh(
        core_axis_name="c", subcore_axis_name="s",
        num_cores=NC, num_subcores=NS)


@functools.lru_cache(maxsize=None)
def _make_count_kernel(n, kbp):
    na = _round_up(n + 1, NS * B)
    rpt = na // NS

    @functools.partial(
        pl.kernel,
        out_type=jax.ShapeDtypeStruct((NC, n, L), jnp.float32),
        mesh=_mesh(),
        scratch_types=[
            pltpu.VMEM((kbp, B), jnp.int32),
            pltpu.VMEM((B, L), jnp.float32),
            pltpu.VMEM((B, L), jnp.float32),
            pltpu.VMEM_SHARED((na, L), jnp.float32),
        ],
    )
    def countk(dst_hbm, out_hbm, didx, ones, zer, accum):
        c = lax.axis_index("c")
        s = lax.axis_index("s")
        w = c * NS + s
        pltpu.sync_copy(dst_hbm.at[w], didx)

        @pl.loop(0, B)
        def _(i):
            ones[i, :] = jnp.full((L,), 1.0, jnp.float32)
            zer[i, :] = jnp.zeros((L,), jnp.float32)

        base = s * rpt
        for k in range(rpt // B):
            pltpu.sync_copy(zer, accum.at[pl.ds(base + k * B, B)])
        plsc.subcore_barrier()

        @pl.loop(0, kbp)
        def _(j):
            pltpu.sync_copy(ones, accum.at[didx.at[j]], add=True)
        plsc.subcore_barrier()

        nfull = n // rpt
        rem = n - nfull * rpt

        @pl.when(s < nfull)
        def _():
            pltpu.sync_copy(accum.at[pl.ds(base, rpt)],
                            out_hbm.at[c, pl.ds(base, rpt)])
        if rem:
            @pl.when(s == nfull)
            def _():
                pltpu.sync_copy(accum.at[pl.ds(nfull * rpt, rem)],
                                out_hbm.at[c, pl.ds(nfull * rpt, rem)])

    return countk


@functools.lru_cache(maxsize=None)
def _make_layer_kernel(n, d, kbp):
    na = _round_up(n + 1, NS * B)
    rpt = na // NS

    @functools.partial(
        pl.kernel,
        out_type=jax.ShapeDtypeStruct((NC, n, d), jnp.float32),
        mesh=_mesh(),
        scratch_types=[
            pltpu.VMEM((kbp, B), jnp.int32),
            pltpu.VMEM((kbp, B), jnp.int32),
            pltpu.VMEM((2, B, d), jnp.float32),
            pltpu.VMEM_SHARED((na, d), jnp.float32),
            pltpu.SemaphoreType.DMA,
            pltpu.SemaphoreType.DMA,
        ],
    )
    def layerk(src_hbm, dst_hbm, h_hbm, out_hbm,
               sidx, didx, rows, accum, sem0, sem1):
        c = lax.axis_index("c")
        s = lax.axis_index("s")
        w = c * NS + s
        pltpu.sync_copy(src_hbm.at[w], sidx)
        pltpu.sync_copy(dst_hbm.at[w], didx)

        nvec = d // L

        @pl.loop(0, B * nvec)
        def _(i):
            r = i // nvec
            cc = i % nvec
            rows[0, r, pl.ds(cc * L, L)] = jnp.zeros((L,), jnp.float32)

        base = s * rpt
        for k in range(rpt // B):
            pltpu.sync_copy(rows.at[0], accum.at[pl.ds(base + k * B, B)])
        plsc.subcore_barrier()

        sems = (sem0, sem1)

        def g_start(j, b):
            pltpu.async_copy(h_hbm.at[sidx.at[j]], rows.at[b], sems[b])

        def g_wait(j, b):
            pltpu.make_async_copy(h_hbm.at[sidx.at[j]], rows.at[b],
                                  sems[b]).wait()

        g_start(0, 0)

        @pl.loop(0, kbp, step=2)
        def _(jj):
            for bb in range(2):
                j = jj + bb
                nxt = j + 1

                @pl.when(nxt < kbp)
                def _():
                    g_start(nxt, (bb + 1) % 2)
                g_wait(j, bb)
                pltpu.sync_copy(rows.at[bb], accum.at[didx.at[j]], add=True)
        plsc.subcore_barrier()

        nfull = n // rpt
        rem = n - nfull * rpt

        @pl.when(s < nfull)
        def _():
            pltpu.sync_copy(accum.at[pl.ds(base, rpt)],
                            out_hbm.at[c, pl.ds(base, rpt)])
        if rem:
            @pl.when(s == nfull)
            def _():
                pltpu.sync_copy(accum.at[pl.ds(nfull * rpt, rem)],
                                out_hbm.at[c, pl.ds(nfull * rpt, rem)])

    return layerk


def _combine(partials, counts, n, d):
    rb = 1000
    assert n % rb == 0

    def body(p_ref, c_ref, o_ref):
        ssum = p_ref[0] + p_ref[1]
        cnt = c_ref[0, :, 0:1] + c_ref[1, :, 0:1]
        o_ref[...] = ssum / jnp.maximum(cnt, 1.0)

    return pl.pallas_call(
        body,
        grid=(n // rb,),
        in_specs=[
            pl.BlockSpec((NC, rb, d), lambda i: (0, i, 0)),
            pl.BlockSpec((NC, rb, L), lambda i: (0, i, 0)),
        ],
        out_specs=pl.BlockSpec((rb, d), lambda i: (i, 0)),
        out_shape=jax.ShapeDtypeStruct((n, d), jnp.float32),
    )(partials, counts)


def kernel(x, edge_index):
    n, d = x.shape
    e = edge_index.shape[1]
    src = edge_index[0]
    dst = edge_index[1]

    ew = e // NW
    assert ew * NW == e
    kbp = -(-ew // B)
    if kbp % 2:
        kbp += 1
    padn = kbp * B - ew
    src_p = jnp.pad(src.reshape(NW, ew), ((0, 0), (0, padn))
                    ).reshape(NW, kbp, B)
    dst_p = jnp.pad(dst.reshape(NW, ew), ((0, 0), (0, padn)),
                    constant_values=n).reshape(NW, kbp, B)

    counts = _make_count_kernel(n, kbp)(dst_p)
    layerk = _make_layer_kernel(n, d, kbp)
    h = x
    for _ in range(NUM_LAYERS):
        partials = layerk(src_p, dst_p, h)
        h = _combine(partials, counts, n, d)
    return h


# speedup vs baseline: 4.1833x; 4.1833x over previous
import functools

import jax
import jax.numpy as jnp
from jax import lax
from jax.experimental import pallas as pl
from jax.experimental.pallas import tpu as pltpu
from jax.experimental.pallas import tpu_sc as plsc

NC = 2
NS = 16
L = 16
NW = NC * NS
B = 128
NUM_LAYERS = 3


def _round_up(v, m):
    return (v + m - 1) // m * m


def _mesh():
    return plsc.VectorSubcoreMesh(
        core_axis_name="c", subcore_axis_name="s",
        num_cores=NC, num_subcores=NS)


@functools.lru_cache(maxsize=None)
def _make_layer_kernel(n, d, kbp):
    na = _round_up(n + 1, NS * B)
    rpt = na // NS

    @functools.partial(
        pl.kernel,
        out_type=jax.ShapeDtypeStruct((NC, n, d), jnp.float32),
        mesh=_mesh(),
        scratch_types=[
            pltpu.VMEM((kbp, B), jnp.int32),
            pltpu.VMEM((kbp, B), jnp.int32),
            pltpu.VMEM((B, d), jnp.float32),
            pltpu.VMEM_SHARED((na, d), jnp.float32),
        ],
    )
    def layerk(src_hbm, dst_hbm, h_hbm, out_hbm,
               sidx, didx, rows, accum):
        c = lax.axis_index("c")
        s = lax.axis_index("s")
        w = c * NS + s
        pltpu.sync_copy(src_hbm.at[w], sidx)
        pltpu.sync_copy(dst_hbm.at[w], didx)

        nvec = d // L

        @pl.loop(0, B * nvec)
        def _(i):
            r = i // nvec
            cc = i % nvec
            rows[r, pl.ds(cc * L, L)] = jnp.zeros((L,), jnp.float32)

        base = s * rpt
        for k in range(rpt // B):
            pltpu.sync_copy(rows, accum.at[pl.ds(base + k * B, B)])
        plsc.subcore_barrier()

        @pl.loop(0, kbp)
        def _(j):
            pltpu.sync_copy(h_hbm.at[sidx.at[j]], rows)
            pltpu.sync_copy(rows, accum.at[didx.at[j]], add=True)
        plsc.subcore_barrier()

        nfull = n // rpt
        rem = n - nfull * rpt

        @pl.when(s < nfull)
        def _():
            pltpu.sync_copy(accum.at[pl.ds(base, rpt)],
                            out_hbm.at[c, pl.ds(base, rpt)])
        if rem:
            @pl.when(s == nfull)
            def _():
                pltpu.sync_copy(accum.at[pl.ds(nfull * rpt, rem)],
                                out_hbm.at[c, pl.ds(nfull * rpt, rem)])

    return layerk


def _combine(partials, counts, n, d):
    rb = 1000
    assert n % rb == 0

    def body(p_ref, c_ref, o_ref):
        ssum = p_ref[0] + p_ref[1]
        cnt = c_ref[0, :, 0:1] + c_ref[1, :, 0:1]
        o_ref[...] = ssum / jnp.maximum(cnt, 1.0)

    return pl.pallas_call(
        body,
        grid=(n // rb,),
        in_specs=[
            pl.BlockSpec((NC, rb, d), lambda i: (0, i, 0)),
            pl.BlockSpec((NC, rb, d), lambda i: (0, i, 0)),
        ],
        out_specs=pl.BlockSpec((rb, d), lambda i: (i, 0)),
        out_shape=jax.ShapeDtypeStruct((n, d), jnp.float32),
    )(partials, counts)


def kernel(x, edge_index):
    n, d = x.shape
    e = edge_index.shape[1]
    src = edge_index[0]
    dst = edge_index[1]

    ew = e // NW
    assert ew * NW == e
    kbp = -(-ew // B)
    padn = kbp * B - ew
    src_p = jnp.pad(src.reshape(NW, ew), ((0, 0), (0, padn))
                    ).reshape(NW, kbp, B)
    dst_p = jnp.pad(dst.reshape(NW, ew), ((0, 0), (0, padn)),
                    constant_values=n).reshape(NW, kbp, B)

    layerk = _make_layer_kernel(n, d, kbp)
    counts = layerk(src_p, dst_p, jnp.ones((n, d), jnp.float32))
    h = x
    for _ in range(NUM_LAYERS):
        partials = layerk(src_p, dst_p, h)
        h = _combine(partials, counts, n, d)
    return h
